# baseline (speedup 1.0000x reference)
import jax
import jax.numpy as jnp
from jax.experimental import pallas as pl
from jax.experimental.pallas import tpu as pltpu
from jax.experimental.pallas import tpu_sc as plsc

N = 10000
E = 320000
D = 128
H = 64
C = 16
THRESH = 0.5

NC = 2
NS = 16
LANES = 16
NW = NC * NS
EPW = E // NW
PK = 128
GR = 10112
DUMMY = N
RPS = GR // NS
PADB = 10544
NBUF = 4

_mesh = plsc.VectorSubcoreMesh(core_axis_name="c", subcore_axis_name="s")
_params = pltpu.CompilerParams(use_tc_tiling_on_sc=False,
                               needs_layout_passes=False)


def _sc_compact(mask, hv, vv, hrow, hcol, vrow, vcol, ni):
    @pl.kernel(
        out_type=(
            jax.ShapeDtypeStruct((E,), jnp.float32),
            jax.ShapeDtypeStruct((E,), jnp.float32),
            jax.ShapeDtypeStruct((NW * PADB,), jnp.int32),
            jax.ShapeDtypeStruct((NW * PADB,), jnp.int32),
            jax.ShapeDtypeStruct((NW * PADB,), jnp.int32),
            jax.ShapeDtypeStruct((NW * 32,), jnp.int32),
        ),
        mesh=_mesh,
        scratch_types=[
            pltpu.VMEM((EPW,), jnp.float32),
            pltpu.VMEM((EPW,), jnp.float32),
            pltpu.VMEM((EPW,), jnp.float32),
            pltpu.VMEM((EPW,), jnp.int32),
            pltpu.VMEM((EPW,), jnp.int32),
            pltpu.VMEM((EPW,), jnp.int32),
            pltpu.VMEM((EPW,), jnp.int32),
            pltpu.VMEM((LANES,), jnp.int32),
            pltpu.VMEM((PADB,), jnp.int32),
            pltpu.VMEM((PADB,), jnp.int32),
            pltpu.VMEM((PADB,), jnp.int32),
            pltpu.VMEM((32,), jnp.int32),
        ],
        compiler_params=_params,
    )
    def k(mask_h, hv_h, vv_h, hrow_h, hcol_h, vrow_h, vcol_h, ni_h,
          mh_h, mv_h, gidx_h, sidx_h, hidx_h, cnt_h,
          m_v, hv_v, vv_v, hr_v, hc_v, vr_v, vc_v, ni_v,
          gidx_v, sidx_v, hidx_v, cnt_v):
        cid = jax.lax.axis_index("c")
        sid = jax.lax.axis_index("s")
        wid = sid * NC + cid
        base = wid * EPW
        pltpu.sync_copy(mask_h.at[pl.ds(base, EPW)], m_v)
        pltpu.sync_copy(hv_h.at[pl.ds(base, EPW)], hv_v)
        pltpu.sync_copy(vv_h.at[pl.ds(base, EPW)], vv_v)
        pltpu.sync_copy(hrow_h.at[pl.ds(base, EPW)], hr_v)
        pltpu.sync_copy(hcol_h.at[pl.ds(base, EPW)], hc_v)
        pltpu.sync_copy(vrow_h.at[pl.ds(base, EPW)], vr_v)
        pltpu.sync_copy(vcol_h.at[pl.ds(base, EPW)], vc_v)
        pltpu.sync_copy(ni_h, ni_v)

        niv = ni_v[...]
        zi = jnp.zeros((LANES,), jnp.int32)

        @pl.loop(0, EPW // LANES, init_carry=(zi, zi), unroll=4)
        def counts(t, carry):
            vcnt, hcnt = carry
            off = t * LANES
            m = m_v[pl.ds(off, LANES)]
            sig = 1.0 / (1.0 + jnp.exp(-m))
            mh = hv_v[pl.ds(off, LANES)] * sig
            mv = vv_v[pl.ds(off, LANES)] * sig
            hv_v[pl.ds(off, LANES)] = mh
            vv_v[pl.ds(off, LANES)] = mv
            av = mv > THRESH
            vco = vcnt[0]
            plsc.store_compressed(gidx_v.at[pl.ds(vco, LANES)],
                                  vc_v[pl.ds(off, LANES)], mask=av)
            plsc.store_compressed(sidx_v.at[pl.ds(vco, LANES)],
                                  vr_v[pl.ds(off, LANES)], mask=av)
            ah = (mh > THRESH) & (hr_v[pl.ds(off, LANES)] == niv)
            hco = hcnt[0]
            plsc.store_compressed(hidx_v.at[pl.ds(hco, LANES)],
                                  hc_v[pl.ds(off, LANES)], mask=ah)
            return (vcnt + plsc.all_reduce_population_count(av),
                    hcnt + plsc.all_reduce_population_count(ah))

        vcnt, hcnt = counts
        vcs = vcnt[0]
        hcs = hcnt[0]

        cnt_v[pl.ds(0, LANES)] = vcnt
        cnt_v[pl.ds(LANES, LANES)] = hcnt

        lane = jax.lax.iota(jnp.int32, LANES)
        dmy = jnp.full((LANES,), DUMMY, jnp.int32)

        @pl.loop(0, (NBUF * PK + 2 * LANES) // LANES)
        def _(t):
            vo = (vcs // LANES) * LANES + t * LANES
            gl = vo + lane
            gidx_v[pl.ds(vo, LANES)] = jnp.where(
                gl >= vcs, zi, gidx_v[pl.ds(vo, LANES)])
            sidx_v[pl.ds(vo, LANES)] = jnp.where(
                gl >= vcs, dmy, sidx_v[pl.ds(vo, LANES)])

        @pl.loop(0, (PK + 2 * LANES) // LANES)
        def _(t):
            ho = (hcs // LANES) * LANES + t * LANES
            hidx_v[pl.ds(ho, LANES)] = jnp.where(
                ho + lane >= hcs, dmy, hidx_v[pl.ds(ho, LANES)])

        obase = wid * PADB
        pltpu.sync_copy(hv_v, mh_h.at[pl.ds(base, EPW)])
        pltpu.sync_copy(vv_v, mv_h.at[pl.ds(base, EPW)])
        pltpu.sync_copy(gidx_v, gidx_h.at[pl.ds(obase, PADB)])
        pltpu.sync_copy(sidx_v, sidx_h.at[pl.ds(obase, PADB)])
        pltpu.sync_copy(hidx_v, hidx_h.at[pl.ds(obase, PADB)])
        pltpu.sync_copy(cnt_v, cnt_h.at[pl.ds(wid * 32, 32)])

    return k(mask, hv, vv, hrow, hcol, vrow, vcol, ni)


def _sc_scatter(xw1, gidx, sidx, hidx, cnt):
    @pl.kernel(
        out_type=(
            jax.ShapeDtypeStruct((NC, GR, H), jnp.float32),
            jax.ShapeDtypeStruct((NC * GR,), jnp.float32),
        ),
        mesh=_mesh,
        scratch_types=[
            pltpu.VMEM((PADB,), jnp.int32),
            pltpu.VMEM((PADB,), jnp.int32),
            pltpu.VMEM((PADB,), jnp.int32),
            pltpu.VMEM((32,), jnp.int32),
            pltpu.VMEM((NBUF, PK), jnp.int32),
            pltpu.VMEM((1, PK), jnp.int32),
            pltpu.VMEM((NBUF, PK, H), jnp.float32),
            pltpu.VMEM((PK,), jnp.float32),
            pltpu.VMEM((PK, H), jnp.float32),
            pltpu.VMEM((RPS + 8,), jnp.float32),
            pltpu.VMEM_SHARED((GR, H), jnp.float32),
            pltpu.VMEM_SHARED((GR,), jnp.float32),
            pltpu.SemaphoreType.DMA,
            pltpu.SemaphoreType.DMA,
        ],
        compiler_params=_params,
    )
    def k(xw1_h, gidx_h, sidx_h, hidx_h, cnt_h, g_h, c_h,
          gidx_v, sidx_v, hidx_v, cnt_v, s2d, h2d, gbuf, ones_v, zblk, z1d,
          g_s, c_s, sem, gsem):
        cid = jax.lax.axis_index("c")
        sid = jax.lax.axis_index("s")
        wid = sid * NC + cid
        obase = wid * PADB

        pltpu.sync_copy(cnt_h.at[pl.ds(wid * 32, 32)], cnt_v)
        vcs = cnt_v[pl.ds(0, LANES)][0]
        hcs = cnt_v[pl.ds(LANES, LANES)][0]
        nv = (vcs + (PK - 1)) // PK
        nh = (hcs + (PK - 1)) // PK
        pltpu.sync_copy(gidx_h.at[pl.ds(obase, PADB)], gidx_v)
        pltpu.sync_copy(sidx_h.at[pl.ds(obase, PADB)], sidx_v)
        pltpu.sync_copy(hidx_h.at[pl.ds(obase, PADB)], hidx_v)

        zf = jnp.zeros((LANES,), jnp.float32)
        of = jnp.ones((LANES,), jnp.float32)

        @pl.loop(0, PK // LANES)
        def _(t):
            ones_v[pl.ds(t * LANES, LANES)] = of

        @pl.loop(0, PK)
        def _(r):
            @pl.loop(0, H // LANES)
            def _(t):
                zblk[r, pl.ds(t * LANES, LANES)] = zf

        @pl.loop(0, (RPS + 8) // LANES)
        def _(t):
            z1d[pl.ds(t * LANES, LANES)] = zf

        r0 = sid * RPS
        for kk in range(RPS // PK):
            pltpu.sync_copy(zblk, g_s.at[pl.ds(r0 + kk * PK, PK)])
        rem = RPS - (RPS // PK) * PK
        pltpu.sync_copy(zblk.at[pl.ds(0, rem)],
                        g_s.at[pl.ds(r0 + (RPS // PK) * PK, rem)])
        pltpu.sync_copy(z1d.at[pl.ds(0, RPS)], c_s.at[pl.ds(r0, RPS)])

        plsc.subcore_barrier()

        @pl.loop(0, nv)
        def _(j):
            @pl.loop(0, PK // LANES)
            def _(t):
                s2d[0, pl.ds(t * LANES, LANES)] = (
                    sidx_v[pl.ds(j * PK + t * LANES, LANES)])
            pltpu.async_copy(
                xw1_h.at[gidx_v.at[pl.ds(j * PK, PK)]], gbuf.at[0],
                gsem).wait()
            pltpu.sync_copy(gbuf.at[0], g_s.at[s2d.at[0]], add=True)

        @pl.loop(0, nh)
        def _(j):
            @pl.loop(0, PK // LANES)
            def _(t):
                h2d[0, pl.ds(t * LANES, LANES)] = (
                    hidx_v[pl.ds(j * PK + t * LANES, LANES)])
            pltpu.sync_copy(ones_v, c_s.at[h2d.at[0]], add=True)

        plsc.subcore_barrier()

        pltpu.sync_copy(g_s.at[pl.ds(r0, RPS)], g_h.at[cid, pl.ds(r0, RPS)])
        pltpu.sync_copy(c_s.at[pl.ds(r0, RPS)],
                        c_h.at[pl.ds(cid * GR + r0, RPS)])

    return k(xw1, gidx, sidx, hidx, cnt)


def _tc_matmul(x, w1):
    def body(x_ref, w_ref, o_ref):
        o_ref[...] = jnp.dot(x_ref[...], w_ref[...],
                             preferred_element_type=jnp.float32)

    return pl.pallas_call(
        body,
        out_shape=jax.ShapeDtypeStruct((N, H), jnp.float32),
    )(x, w1)


def _tc_finish(g, c, w2):
    def body(g_ref, c_ref, w2_ref, o_ref):
        gg = jnp.maximum(g_ref[0, :N, :] + g_ref[1, :N, :], 0.0)
        cc = c_ref[0, :N, :] + c_ref[1, :N, :]
        acc = jnp.sum(gg * cc, axis=0, keepdims=True)
        y = jnp.dot(acc, w2_ref[...], preferred_element_type=jnp.float32)
        m = jnp.max(y, axis=1, keepdims=True)
        e = jnp.exp(y - m)
        o_ref[...] = e / jnp.sum(e, axis=1, keepdims=True)

    return pl.pallas_call(
        body,
        out_shape=jax.ShapeDtypeStruct((1, C), jnp.float32),
    )(g, c, w2)


def kernel(mask, hor_indices, hor_values, ver_indices, ver_values,
           X, W1, W2, node_idx):
    hrow = hor_indices[0]
    hcol = hor_indices[1]
    vrow = ver_indices[0]
    vcol = ver_indices[1]
    ni = jnp.full((LANES,), node_idx, jnp.int32)

    mh, mv, gidx, sidx, hidx, cnt = _sc_compact(
        mask, hor_values, ver_values, hrow, hcol, vrow, vcol, ni)
    xw1 = _tc_matmul(X, W1)
    g, c = _sc_scatter(xw1, gidx, sidx, hidx, cnt)
    res = _tc_finish(g, c.reshape(NC, GR, 1), W2)
    return (res.reshape(C), mh, mv)


# speedup vs baseline: 1.2402x; 1.2402x over previous
import jax
import jax.numpy as jnp
from jax.experimental import pallas as pl
from jax.experimental.pallas import tpu as pltpu
from jax.experimental.pallas import tpu_sc as plsc

N = 10000
E = 320000
D = 128
H = 64
C = 16
THRESH = 0.5

NC = 2
NS = 16
LANES = 16
NW = NC * NS
PK = 128
EPW_A = 10112
EPW_L = E - (NW - 1) * EPW_A
GR = 10112
DUMMY = N
RPS = GR // NS
PADW = 4 * PK + 2 * LANES
PADB = EPW_A + PADW

_mesh = plsc.VectorSubcoreMesh(core_axis_name="c", subcore_axis_name="s")
_params = pltpu.CompilerParams(use_tc_tiling_on_sc=False,
                               needs_layout_passes=False)


def _sc_compact(mask, hv, vv, hor, ver, ni):
    @pl.kernel(
        out_type=(
            jax.ShapeDtypeStruct((E,), jnp.float32),
            jax.ShapeDtypeStruct((E,), jnp.float32),
            jax.ShapeDtypeStruct((NW * PADB,), jnp.int32),
            jax.ShapeDtypeStruct((NW * PADB,), jnp.int32),
            jax.ShapeDtypeStruct((NW * PADB,), jnp.int32),
            jax.ShapeDtypeStruct((NW * 32,), jnp.int32),
        ),
        mesh=_mesh,
        scratch_types=[
            pltpu.VMEM((EPW_A,), jnp.float32),
            pltpu.VMEM((EPW_A,), jnp.float32),
            pltpu.VMEM((EPW_A,), jnp.float32),
            pltpu.VMEM((2, EPW_A), jnp.int32),
            pltpu.VMEM((2, EPW_A), jnp.int32),
            pltpu.VMEM((LANES,), jnp.int32),
            pltpu.VMEM((PADB,), jnp.int32),
            pltpu.VMEM((PADB,), jnp.int32),
            pltpu.VMEM((PADB,), jnp.int32),
            pltpu.VMEM((32,), jnp.int32),
        ],
        compiler_params=_params,
    )
    def k(mask_h, hv_h, vv_h, hor_h, ver_h, ni_h,
          mh_h, mv_h, gidx_h, sidx_h, hidx_h, cnt_h,
          m_v, hv_v, vv_v, hi_v, vi_v, ni_v,
          gidx_v, sidx_v, hidx_v, cnt_v):
        cid = jax.lax.axis_index("c")
        sid = jax.lax.axis_index("s")
        wid = sid * NC + cid
        base = wid * EPW_A
        last = wid == NW - 1
        epw = jnp.where(last, EPW_L, EPW_A)

        @pl.when(jnp.logical_not(last))
        def _():
            pltpu.sync_copy(mask_h.at[pl.ds(base, EPW_A)], m_v)
            pltpu.sync_copy(hv_h.at[pl.ds(base, EPW_A)], hv_v)
            pltpu.sync_copy(vv_h.at[pl.ds(base, EPW_A)], vv_v)
            pltpu.sync_copy(hor_h.at[pl.ds(0, 2), pl.ds(base, EPW_A)], hi_v)
            pltpu.sync_copy(ver_h.at[pl.ds(0, 2), pl.ds(base, EPW_A)], vi_v)

        @pl.when(last)
        def _():
            pltpu.sync_copy(mask_h.at[pl.ds(base, EPW_L)],
                            m_v.at[pl.ds(0, EPW_L)])
            pltpu.sync_copy(hv_h.at[pl.ds(base, EPW_L)],
                            hv_v.at[pl.ds(0, EPW_L)])
            pltpu.sync_copy(vv_h.at[pl.ds(base, EPW_L)],
                            vv_v.at[pl.ds(0, EPW_L)])
            pltpu.sync_copy(hor_h.at[pl.ds(0, 2), pl.ds(base, EPW_L)],
                            hi_v.at[pl.ds(0, 2), pl.ds(0, EPW_L)])
            pltpu.sync_copy(ver_h.at[pl.ds(0, 2), pl.ds(base, EPW_L)],
                            vi_v.at[pl.ds(0, 2), pl.ds(0, EPW_L)])

        pltpu.sync_copy(ni_h, ni_v)

        niv = ni_v[...]
        zi = jnp.zeros((LANES,), jnp.int32)

        @pl.loop(0, epw // LANES, init_carry=(zi, zi))
        def counts(t, carry):
            vcnt, hcnt = carry
            off = t * LANES
            m = m_v[pl.ds(off, LANES)]
            sig = 1.0 / (1.0 + jnp.exp(-m))
            mh = hv_v[pl.ds(off, LANES)] * sig
            mv = vv_v[pl.ds(off, LANES)] * sig
            hv_v[pl.ds(off, LANES)] = mh
            vv_v[pl.ds(off, LANES)] = mv
            av = mv > THRESH
            vco = vcnt[0]
            plsc.store_compressed(gidx_v.at[pl.ds(vco, LANES)],
                                  vi_v[1, pl.ds(off, LANES)], mask=av)
            plsc.store_compressed(sidx_v.at[pl.ds(vco, LANES)],
                                  vi_v[0, pl.ds(off, LANES)], mask=av)
            ah = (mh > THRESH) & (hi_v[0, pl.ds(off, LANES)] == niv)
            hco = hcnt[0]
            plsc.store_compressed(hidx_v.at[pl.ds(hco, LANES)],
                                  hi_v[1, pl.ds(off, LANES)], mask=ah)
            return (vcnt + plsc.all_reduce_population_count(av),
                    hcnt + plsc.all_reduce_population_count(ah))

        vcnt, hcnt = counts
        vcs = vcnt[0]
        hcs = hcnt[0]

        cnt_v[pl.ds(0, LANES)] = vcnt
        cnt_v[pl.ds(LANES, LANES)] = hcnt

        lane = jax.lax.iota(jnp.int32, LANES)
        dmy = jnp.full((LANES,), DUMMY, jnp.int32)

        @pl.loop(0, PADW // LANES)
        def _(t):
            vo = (vcs // LANES) * LANES + t * LANES
            gl = vo + lane
            gidx_v[pl.ds(vo, LANES)] = jnp.where(
                gl >= vcs, zi, gidx_v[pl.ds(vo, LANES)])
            sidx_v[pl.ds(vo, LANES)] = jnp.where(
                gl >= vcs, dmy, sidx_v[pl.ds(vo, LANES)])

        @pl.loop(0, (PK + 2 * LANES) // LANES)
        def _(t):
            ho = (hcs // LANES) * LANES + t * LANES
            hidx_v[pl.ds(ho, LANES)] = jnp.where(
                ho + lane >= hcs, dmy, hidx_v[pl.ds(ho, LANES)])

        obase = wid * PADB

        @pl.when(jnp.logical_not(last))
        def _():
            pltpu.sync_copy(hv_v, mh_h.at[pl.ds(base, EPW_A)])
            pltpu.sync_copy(vv_v, mv_h.at[pl.ds(base, EPW_A)])

        @pl.when(last)
        def _():
            pltpu.sync_copy(hv_v.at[pl.ds(0, EPW_L)],
                            mh_h.at[pl.ds(base, EPW_L)])
            pltpu.sync_copy(vv_v.at[pl.ds(0, EPW_L)],
                            mv_h.at[pl.ds(base, EPW_L)])

        pltpu.sync_copy(gidx_v, gidx_h.at[pl.ds(obase, PADB)])
        pltpu.sync_copy(sidx_v, sidx_h.at[pl.ds(obase, PADB)])
        pltpu.sync_copy(hidx_v, hidx_h.at[pl.ds(obase, PADB)])
        pltpu.sync_copy(cnt_v, cnt_h.at[pl.ds(wid * 32, 32)])

    return k(mask, hv, vv, hor, ver, ni)


def _sc_scatter(xw1, gidx, sidx, hidx, cnt, zrows, zc, ones):
    @pl.kernel(
        out_type=(
            jax.ShapeDtypeStruct((NC, GR, H), jnp.float32),
            jax.ShapeDtypeStruct((NC * GR,), jnp.float32),
        ),
        mesh=_mesh,
        scratch_types=[
            pltpu.VMEM((PADB,), jnp.int32),
            pltpu.VMEM((PADB,), jnp.int32),
            pltpu.VMEM((PADB,), jnp.int32),
            pltpu.VMEM((32,), jnp.int32),
            pltpu.VMEM((1, PK), jnp.int32),
            pltpu.VMEM((1, PK), jnp.int32),
            pltpu.VMEM((PK, H), jnp.float32),
            pltpu.VMEM((PK,), jnp.float32),
            pltpu.VMEM_SHARED((GR, H), jnp.float32),
            pltpu.VMEM_SHARED((GR,), jnp.float32),
            pltpu.SemaphoreType.DMA,
        ],
        compiler_params=_params,
    )
    def k(xw1_h, gidx_h, sidx_h, hidx_h, cnt_h, zr_h, zc_h, on_h, g_h, c_h,
          gidx_v, sidx_v, hidx_v, cnt_v, s2d, h2d, gbuf, ones_v,
          g_s, c_s, gsem):
        cid = jax.lax.axis_index("c")
        sid = jax.lax.axis_index("s")
        wid = sid * NC + cid
        obase = wid * PADB

        pltpu.sync_copy(cnt_h.at[pl.ds(wid * 32, 32)], cnt_v)
        vcs = cnt_v[pl.ds(0, LANES)][0]
        hcs = cnt_v[pl.ds(LANES, LANES)][0]
        nv = (vcs + (PK - 1)) // PK
        nh = (hcs + (PK - 1)) // PK
        pltpu.sync_copy(gidx_h.at[pl.ds(obase, PADB)], gidx_v)
        pltpu.sync_copy(sidx_h.at[pl.ds(obase, PADB)], sidx_v)
        pltpu.sync_copy(hidx_h.at[pl.ds(obase, PADB)], hidx_v)
        pltpu.sync_copy(on_h, ones_v)

        r0 = sid * RPS
        pltpu.sync_copy(zr_h, g_s.at[pl.ds(r0, RPS)])
        pltpu.sync_copy(zc_h.at[pl.ds(0, RPS)], c_s.at[pl.ds(r0, RPS)])

        plsc.subcore_barrier()

        @pl.loop(0, nv)
        def _(j):
            @pl.loop(0, PK // LANES)
            def _(t):
                s2d[0, pl.ds(t * LANES, LANES)] = (
                    sidx_v[pl.ds(j * PK + t * LANES, LANES)])
            pltpu.async_copy(
                xw1_h.at[gidx_v.at[pl.ds(j * PK, PK)]], gbuf, gsem).wait()
            pltpu.sync_copy(gbuf, g_s.at[s2d.at[0]], add=True)

        @pl.loop(0, nh)
        def _(j):
            @pl.loop(0, PK // LANES)
            def _(t):
                h2d[0, pl.ds(t * LANES, LANES)] = (
                    hidx_v[pl.ds(j * PK + t * LANES, LANES)])
            pltpu.sync_copy(ones_v, c_s.at[h2d.at[0]], add=True)

        plsc.subcore_barrier()

        pltpu.sync_copy(g_s.at[pl.ds(r0, RPS)], g_h.at[cid, pl.ds(r0, RPS)])
        pltpu.sync_copy(c_s.at[pl.ds(r0, RPS)],
                        c_h.at[pl.ds(cid * GR + r0, RPS)])

    return k(xw1, gidx, sidx, hidx, cnt, zrows, zc, ones)


def _tc_matmul(x, w1):
    def body(x_ref, w_ref, o_ref):
        o_ref[...] = jnp.dot(x_ref[...], w_ref[...],
                             preferred_element_type=jnp.float32)

    return pl.pallas_call(
        body,
        out_shape=jax.ShapeDtypeStruct((N, H), jnp.float32),
    )(x, w1)


def _tc_finish(g, c, w2):
    def body(g_ref, c_ref, w2_ref, o_ref):
        gg = jnp.maximum(g_ref[0, :N, :] + g_ref[1, :N, :], 0.0)
        cc = (c_ref[pl.ds(0, N)] + c_ref[pl.ds(GR, N)]).reshape(1, N)
        acc = jnp.dot(cc, gg, preferred_element_type=jnp.float32)
        y = jnp.dot(acc, w2_ref[...], preferred_element_type=jnp.float32)
        m = jnp.max(y, axis=1, keepdims=True)
        e = jnp.exp(y - m)
        o_ref[...] = e / jnp.sum(e, axis=1, keepdims=True)

    return pl.pallas_call(
        body,
        out_shape=jax.ShapeDtypeStruct((1, C), jnp.float32),
    )(g, c, w2)


def kernel(mask, hor_indices, hor_values, ver_indices, ver_values,
           X, W1, W2, node_idx):
    ni = jnp.full((LANES,), node_idx, jnp.int32)
    zrows = jnp.zeros((RPS, H), jnp.float32)
    zc = jnp.zeros((RPS + 8,), jnp.float32)
    ones = jnp.ones((PK,), jnp.float32)

    mh, mv, gidx, sidx, hidx, cnt = _sc_compact(
        mask, hor_values, ver_values, hor_indices, ver_indices, ni)
    xw1 = _tc_matmul(X, W1)
    g, c = _sc_scatter(xw1, gidx, sidx, hidx, cnt, zrows, zc, ones)
    res = _tc_finish(g, c, W2)
    return (res.reshape(C), mh, mv)
